# baseline (speedup 1.0000x reference)
import functools

import jax
import jax.numpy as jnp
from jax.experimental import pallas as pl
from jax.experimental.pallas import tpu as pltpu


_M = 4
_W = 128


def _bf16_split(v, n):
    out = []
    for _ in range(n):
        b = v.astype(jnp.bfloat16).astype(jnp.float32)
        out.append(b)
        v = v - b
    return out


def _drq_kernel(scale_ref, x_ref, cb_ref, codes_ref, loss_ref,
                tab_ref, cn_ref):
    i = pl.program_id(0)
    k, d = cb_ref.shape

    @pl.when(i == 0)
    def _init():
        cb = cb_ref[...]
        pad = jnp.zeros((k, _W - d - 3), jnp.float32)
        ones = jnp.ones((k, 1), jnp.float32)
        iota = jax.lax.broadcasted_iota(
            jnp.int32, (k, 1), 0).astype(jnp.float32)
        i0, i1 = _bf16_split(iota, 2)
        for m in range(_M):
            cbm = cb * scale_ref[m]
            cn = jnp.sum(cbm * cbm, axis=1)
            cn_ref[m] = jnp.broadcast_to(cn[None, :], (8, k))
            tab_ref[m] = jnp.concatenate(
                [cbm, ones, i0, i1, pad], axis=1).astype(jnp.bfloat16)
        loss_ref[0] = 0.0

    x = x_ref[...]
    bn = x.shape[0]
    n_total = pl.num_programs(0) * bn
    inv_nd = 1.0 / (n_total * d)

    dot = functools.partial(
        jax.lax.dot_general, preferred_element_type=jnp.float32)

    residual = x
    qsoft = jnp.zeros_like(x)
    qhard = jnp.zeros_like(x)
    part = jnp.float32(0.0)
    for m in range(_M):
        tab = tab_ref[m]
        cnl = cn_ref[m, 0:1, :]
        rn = jnp.sum(residual * residual, axis=1, keepdims=True)
        r2 = (residual + residual).astype(jnp.bfloat16)
        g2 = dot(r2, tab[:, :d], (((1,), (1,)), ((), ())))
        nd = (rn - g2) + cnl
        mn = jnp.min(nd, axis=1, keepdims=True)
        e = jnp.exp(mn - nd).astype(jnp.bfloat16)
        se = dot(e, tab, (((1,), (0,)), ((), ())))
        soft = se[:, :d] / se[:, d:d + 1]
        mask = (nd <= mn).astype(jnp.bfloat16)
        hv = dot(mask, tab, (((1,), (0,)), ((), ())))
        cnt = hv[:, d:d + 1]
        code_f = (hv[:, d + 1:d + 2] + hv[:, d + 2:d + 3]) / cnt
        codes_ref[:, m] = code_f[:, 0].astype(jnp.int32)
        hard = hv[:, :d] / cnt
        residual = residual - hard
        qsoft = qsoft + soft
        qhard = qhard + hard
        part += 0.1 * jnp.sum((x - qsoft) ** 2) + jnp.sum((x - qhard) ** 2)
    part += 0.1 * jnp.sum((qsoft - qhard) ** 2)

    loss_ref[0] += part * inv_nd


def kernel(x, codebook, scale):
    n, d = x.shape
    k = codebook.shape[0]
    bn = 512
    grid = (n // bn,)
    codes, loss = pl.pallas_call(
        _drq_kernel,
        grid=grid,
        in_specs=[
            pl.BlockSpec(memory_space=pltpu.SMEM),
            pl.BlockSpec((bn, d), lambda i: (i, 0)),
            pl.BlockSpec((k, d), lambda i: (0, 0)),
        ],
        out_specs=[
            pl.BlockSpec((bn, _M), lambda i: (i, 0)),
            pl.BlockSpec(memory_space=pltpu.SMEM),
        ],
        out_shape=[
            jax.ShapeDtypeStruct((n, _M), jnp.int32),
            jax.ShapeDtypeStruct((1,), jnp.float32),
        ],
        scratch_shapes=[
            pltpu.VMEM((_M, k, _W), jnp.bfloat16),
            pltpu.VMEM((_M, 8, k), jnp.float32),
        ],
        compiler_params=pltpu.CompilerParams(
            dimension_semantics=("arbitrary",)),
    )(scale, x, codebook)
    return codes, loss[0]


# speedup vs baseline: 1.1451x; 1.1451x over previous
import functools

import jax
import jax.numpy as jnp
from jax.experimental import pallas as pl
from jax.experimental.pallas import tpu as pltpu


_M = 4
_W = 128


def _bf16_split(v, n):
    out = []
    for _ in range(n):
        b = v.astype(jnp.bfloat16).astype(jnp.float32)
        out.append(b)
        v = v - b
    return out


def _drq_kernel(scale_ref, x_ref, cb_ref, codes_ref, loss_ref,
                tab_ref, cn_ref):
    i = pl.program_id(0)
    k, d = cb_ref.shape

    @pl.when(i == 0)
    def _init():
        cb = cb_ref[...]
        pad = jnp.zeros((k, _W - d - 3), jnp.float32)
        ones = jnp.ones((k, 1), jnp.float32)
        iota = jax.lax.broadcasted_iota(
            jnp.int32, (k, 1), 0).astype(jnp.float32)
        i0, i1 = _bf16_split(iota, 2)
        for m in range(_M):
            cbm = cb * scale_ref[m]
            cn = jnp.sum(cbm * cbm, axis=1)
            cn_ref[m] = jnp.broadcast_to(cn[None, :], (8, k))
            tab_ref[m] = jnp.concatenate(
                [cbm, ones, i0, i1, pad], axis=1).astype(jnp.bfloat16)
        loss_ref[0] = 0.0

    x = x_ref[...]
    bn = x.shape[0]
    n_total = pl.num_programs(0) * bn
    inv_nd = 1.0 / (n_total * d)

    dot = functools.partial(
        jax.lax.dot_general, preferred_element_type=jnp.float32)

    residual = x
    qsoft = jnp.zeros_like(x)
    qhard = jnp.zeros_like(x)
    part = jnp.float32(0.0)
    for m in range(_M):
        tab = tab_ref[m]
        cnl = cn_ref[m, 0:1, :]
        rn = jnp.sum(residual * residual, axis=1, keepdims=True)
        r2 = (residual + residual).astype(jnp.bfloat16)
        g2 = dot(r2, tab[:, :d], (((1,), (1,)), ((), ())))
        nd = (rn - g2) + cnl
        mn = jnp.min(nd, axis=1, keepdims=True)
        e = jnp.exp(mn - nd).astype(jnp.bfloat16)
        se = dot(e, tab, (((1,), (0,)), ((), ())))
        soft = se[:, :d] / se[:, d:d + 1]
        mask = (nd <= mn).astype(jnp.bfloat16)
        hv = dot(mask, tab, (((1,), (0,)), ((), ())))
        cnt = hv[:, d:d + 1]
        code_f = (hv[:, d + 1:d + 2] + hv[:, d + 2:d + 3]) / cnt
        codes_ref[:, m] = code_f[:, 0].astype(jnp.int32)
        hard = hv[:, :d] / cnt
        residual = residual - hard
        qsoft = qsoft + soft
        qhard = qhard + hard
        part += 0.1 * jnp.sum((x - qsoft) ** 2) + jnp.sum((x - qhard) ** 2)
    part += 0.1 * jnp.sum((qsoft - qhard) ** 2)

    loss_ref[0] += part * inv_nd


def kernel(x, codebook, scale):
    n, d = x.shape
    k = codebook.shape[0]
    bn = 1024
    grid = (n // bn,)
    codes, loss = pl.pallas_call(
        _drq_kernel,
        grid=grid,
        in_specs=[
            pl.BlockSpec(memory_space=pltpu.SMEM),
            pl.BlockSpec((bn, d), lambda i: (i, 0)),
            pl.BlockSpec((k, d), lambda i: (0, 0)),
        ],
        out_specs=[
            pl.BlockSpec((bn, _M), lambda i: (i, 0)),
            pl.BlockSpec(memory_space=pltpu.SMEM),
        ],
        out_shape=[
            jax.ShapeDtypeStruct((n, _M), jnp.int32),
            jax.ShapeDtypeStruct((1,), jnp.float32),
        ],
        scratch_shapes=[
            pltpu.VMEM((_M, k, _W), jnp.bfloat16),
            pltpu.VMEM((_M, 8, k), jnp.float32),
        ],
        compiler_params=pltpu.CompilerParams(
            dimension_semantics=("arbitrary",)),
    )(scale, x, codebook)
    return codes, loss[0]
